# baseline (speedup 1.0000x reference)
import jax
import jax.numpy as jnp
from jax.experimental import pallas as pl
from jax.experimental.pallas import tpu as pltpu

_N = 1000
_B = 256
_RC = 8
_NBUF = 16
_RAHEAD = 8

_CHUNKS = (
    [("c", r, _RC) for r in range(0, _B, _RC)]
    + [("m", r, _RC) for r in range(_B, _N, _RC)]
)
_NCH = len(_CHUNKS)


def _enqueue_kernel(comp_hbm, mem_hbm, out_hbm, buf, rsem, wsem):
    def rd(i, s):
        src, r0, nr = _CHUNKS[i]
        ref = comp_hbm if src == "c" else mem_hbm
        return pltpu.make_async_copy(
            ref.at[pl.ds(r0, nr)], buf.at[s, pl.ds(0, nr)], rsem.at[s])

    def wr(i, s):
        _, r0, nr = _CHUNKS[i]
        return pltpu.make_async_copy(
            buf.at[s, pl.ds(0, nr)], out_hbm.at[pl.ds(r0, nr)], wsem.at[s])

    for i in range(_RAHEAD):
        rd(i, i % _NBUF).start()
    for i in range(_NCH):
        s = i % _NBUF
        rd(i, s).wait()
        wr(i, s).start()
        ni = i + _RAHEAD
        if ni < _NCH:
            ns = ni % _NBUF
            if ni >= _NBUF:
                wr(ni - _NBUF, ns).wait()
            rd(ni, ns).start()
    for i in range(_NCH - _NBUF, _NCH):
        wr(i, i % _NBUF).wait()


def kernel(memory_bank, components):
    comps = jax.lax.stop_gradient(components)
    return pl.pallas_call(
        _enqueue_kernel,
        in_specs=[
            pl.BlockSpec(memory_space=pltpu.MemorySpace.HBM),
            pl.BlockSpec(memory_space=pltpu.MemorySpace.HBM),
        ],
        out_specs=pl.BlockSpec(memory_space=pltpu.MemorySpace.HBM),
        out_shape=jax.ShapeDtypeStruct((_N, 256, 256), memory_bank.dtype),
        scratch_shapes=[
            pltpu.VMEM((_NBUF, _RC, 256, 256), jnp.float32),
            pltpu.SemaphoreType.DMA((_NBUF,)),
            pltpu.SemaphoreType.DMA((_NBUF,)),
        ],
    )(comps, memory_bank)


# speedup vs baseline: 1.0048x; 1.0048x over previous
import jax
import jax.numpy as jnp
from jax.experimental import pallas as pl
from jax.experimental.pallas import tpu as pltpu

_N = 1000
_B = 256
_RC = 32
_NBUF = 6
_RAHEAD = 3

_CHUNKS = (
    [("c", r, _RC) for r in range(0, _B, _RC)]
    + [("m", r, _RC) for r in range(_B, _N - 8, _RC)]
    + [("m", _N - 8, 8)]
)
_NCH = len(_CHUNKS)


def _enqueue_kernel(comp_hbm, mem_hbm, out_hbm, buf, rsem, wsem):
    def rd(i, s):
        src, r0, nr = _CHUNKS[i]
        ref = comp_hbm if src == "c" else mem_hbm
        return pltpu.make_async_copy(
            ref.at[pl.ds(r0, nr)], buf.at[s, pl.ds(0, nr)], rsem.at[s])

    def wr(i, s):
        _, r0, nr = _CHUNKS[i]
        return pltpu.make_async_copy(
            buf.at[s, pl.ds(0, nr)], out_hbm.at[pl.ds(r0, nr)], wsem.at[s])

    for i in range(_RAHEAD):
        rd(i, i % _NBUF).start()
    for i in range(_NCH):
        s = i % _NBUF
        rd(i, s).wait()
        wr(i, s).start()
        ni = i + _RAHEAD
        if ni < _NCH:
            ns = ni % _NBUF
            if ni >= _NBUF:
                wr(ni - _NBUF, ns).wait()
            rd(ni, ns).start()
    for i in range(_NCH - _NBUF, _NCH):
        wr(i, i % _NBUF).wait()


def kernel(memory_bank, components):
    comps = jax.lax.stop_gradient(components)
    return pl.pallas_call(
        _enqueue_kernel,
        in_specs=[
            pl.BlockSpec(memory_space=pltpu.MemorySpace.HBM),
            pl.BlockSpec(memory_space=pltpu.MemorySpace.HBM),
        ],
        out_specs=pl.BlockSpec(memory_space=pltpu.MemorySpace.HBM),
        out_shape=jax.ShapeDtypeStruct((_N, 256, 256), memory_bank.dtype),
        scratch_shapes=[
            pltpu.VMEM((_NBUF, _RC, 256, 256), jnp.float32),
            pltpu.SemaphoreType.DMA((_NBUF,)),
            pltpu.SemaphoreType.DMA((_NBUF,)),
        ],
    )(comps, memory_bank)
